# Initial kernel scaffold; baseline (speedup 1.0000x reference)
import jax
import jax.numpy as jnp
from jax.experimental import pallas as pl


def kernel(x, edge_index, W1, b1, W2, b2):
    raise NotImplementedError("write your pallas kernel here")


# speedup vs baseline: 14.2134x; 14.2134x over previous
import functools

import jax
import jax.numpy as jnp
from jax import lax
from jax.experimental import pallas as pl
from jax.experimental.pallas import tpu as pltpu
from jax.experimental.pallas import tpu_sc as plsc

N = 10000
E = 320000
D_IN = 128
D_HID = 64

NC = 2
NS = 16
NW = NC * NS
BB = 128
NB = 79
EPT = NB * BB
E_PAD = EPT * NW
N_ACC = 10240
ZSL = N_ACC // BB // NS
R_DUMP = N_ACC // NS
ROW_BLK = 2000


def _mesh():
    return plsc.VectorSubcoreMesh(core_axis_name="c", subcore_axis_name="s")


@functools.partial(
    pl.kernel,
    out_type=jax.ShapeDtypeStruct((NC, N_ACC), jnp.float32),
    mesh=_mesh(),
    scratch_types=[
        pltpu.VMEM((NB, BB), jnp.int32),
        pltpu.VMEM((BB,), jnp.float32),
        pltpu.VMEM((BB,), jnp.float32),
        pltpu.VMEM_SHARED((N_ACC,), jnp.float32),
    ],
)
def _sc_degree(dst_hbm, ones_hbm, zeros_hbm, deg_hbm, dst_v, ones_v, zeros_v, acc):
    cid = lax.axis_index("c")
    sid = lax.axis_index("s")
    wid = sid * NC + cid
    pltpu.sync_copy(dst_hbm.at[wid], dst_v)
    pltpu.sync_copy(ones_hbm, ones_v)
    pltpu.sync_copy(zeros_hbm, zeros_v)
    for k in range(ZSL):
        pltpu.sync_copy(zeros_v, acc.at[pl.ds((sid * ZSL + k) * BB, BB)])
    plsc.subcore_barrier()

    def body(j, carry):
        pltpu.sync_copy(ones_v, acc.at[dst_v.at[j]], add=True)
        return carry

    lax.fori_loop(0, NB, body, 0)
    plsc.subcore_barrier()
    pltpu.sync_copy(acc.at[pl.ds(sid * R_DUMP, R_DUMP)],
                    deg_hbm.at[cid, pl.ds(sid * R_DUMP, R_DUMP)])


def _make_scatter(D):
    @functools.partial(
        pl.kernel,
        out_type=jax.ShapeDtypeStruct((NC, N_ACC, D), jnp.float32),
        mesh=_mesh(),
        compiler_params=pltpu.CompilerParams(use_tc_tiling_on_sc=False),
        scratch_types=[
            pltpu.VMEM((NB, BB), jnp.int32),
            pltpu.VMEM((NB, BB), jnp.int32),
            pltpu.VMEM((BB, D), jnp.float32),
            pltpu.VMEM((BB, D), jnp.float32),
            pltpu.VMEM_SHARED((N_ACC, D), jnp.float32),
            pltpu.SemaphoreType.DMA,
            pltpu.SemaphoreType.DMA,
        ],
    )
    def _scatter(g_hbm, src_hbm, dst_hbm, zrows_hbm, part_hbm,
                 src_v, dst_v, buf0, buf1, acc, sem0, sem1):
        cid = lax.axis_index("c")
        sid = lax.axis_index("s")
        wid = sid * NC + cid
        pltpu.sync_copy(src_hbm.at[wid], src_v)
        pltpu.sync_copy(dst_hbm.at[wid], dst_v)
        pltpu.sync_copy(zrows_hbm, buf0)
        for k in range(ZSL):
            pltpu.sync_copy(buf0, acc.at[pl.ds((sid * ZSL + k) * BB, BB)])
        plsc.subcore_barrier()

        def body(j, carry):
            pltpu.async_copy(g_hbm.at[src_v.at[j]], buf0, sem0).wait()
            pltpu.sync_copy(buf0, acc.at[dst_v.at[j]], add=True)
            return carry

        lax.fori_loop(0, NB, body, 0)
        plsc.subcore_barrier()
        pltpu.sync_copy(acc.at[pl.ds(sid * R_DUMP, R_DUMP)],
                        part_hbm.at[cid, pl.ds(sid * R_DUMP, R_DUMP)])

    return _scatter


_sc_scatter64 = _make_scatter(D_HID)
_sc_scatter128 = _make_scatter(D_IN)


def _dinv_from(degT_ref):
    deg = degT_ref[...].sum(axis=1, keepdims=True) + 1.0
    return lax.rsqrt(jnp.maximum(deg, 1.0))


def _tc1_body(degT_ref, x_ref, W1_ref, g1_ref):
    dinv = _dinv_from(degT_ref)
    g1_ref[...] = jnp.dot(x_ref[...], W1_ref[...],
                          preferred_element_type=jnp.float32) * dinv


def _tc2_body(degT_ref, p_ref, g1_ref, b1_ref, W2_ref, g2_ref):
    dinv = _dinv_from(degT_ref)
    s = p_ref[0] + p_ref[1] + g1_ref[...]
    h = jnp.maximum(dinv * s + b1_ref[...], 0.0)
    g2_ref[...] = jnp.dot(h, W2_ref[...],
                          preferred_element_type=jnp.float32) * dinv


def _tc3_body(degT_ref, q_ref, g2_ref, b2_ref, out_ref):
    dinv = _dinv_from(degT_ref)
    out_ref[...] = dinv * (q_ref[0] + q_ref[1] + g2_ref[...]) + b2_ref[...]


_GRID = (N // ROW_BLK,)

_tc1 = pl.pallas_call(
    _tc1_body,
    grid=_GRID,
    in_specs=[
        pl.BlockSpec((ROW_BLK, 2), lambda i: (i, 0)),
        pl.BlockSpec((ROW_BLK, D_IN), lambda i: (i, 0)),
        pl.BlockSpec((D_IN, D_HID), lambda i: (0, 0)),
    ],
    out_specs=pl.BlockSpec((ROW_BLK, D_HID), lambda i: (i, 0)),
    out_shape=jax.ShapeDtypeStruct((N, D_HID), jnp.float32),
)

_tc2 = pl.pallas_call(
    _tc2_body,
    grid=_GRID,
    in_specs=[
        pl.BlockSpec((ROW_BLK, 2), lambda i: (i, 0)),
        pl.BlockSpec((NC, ROW_BLK, D_HID), lambda i: (0, i, 0)),
        pl.BlockSpec((ROW_BLK, D_HID), lambda i: (i, 0)),
        pl.BlockSpec((1, D_HID), lambda i: (0, 0)),
        pl.BlockSpec((D_HID, D_IN), lambda i: (0, 0)),
    ],
    out_specs=pl.BlockSpec((ROW_BLK, D_IN), lambda i: (i, 0)),
    out_shape=jax.ShapeDtypeStruct((N, D_IN), jnp.float32),
)

_tc3 = pl.pallas_call(
    _tc3_body,
    grid=_GRID,
    in_specs=[
        pl.BlockSpec((ROW_BLK, 2), lambda i: (i, 0)),
        pl.BlockSpec((NC, ROW_BLK, D_IN), lambda i: (0, i, 0)),
        pl.BlockSpec((ROW_BLK, D_IN), lambda i: (i, 0)),
        pl.BlockSpec((1, D_IN), lambda i: (0, 0)),
    ],
    out_specs=pl.BlockSpec((ROW_BLK, D_IN), lambda i: (i, 0)),
    out_shape=jax.ShapeDtypeStruct((N, D_IN), jnp.float32),
)


def kernel(x, edge_index, W1, b1, W2, b2):
    src = edge_index[0].astype(jnp.int32)
    dst = edge_index[1].astype(jnp.int32)
    pad = E_PAD - E
    srcp = jnp.concatenate([src, jnp.zeros((pad,), jnp.int32)]).reshape(NW, NB, BB)
    dstp = jnp.concatenate([dst, jnp.full((pad,), N, jnp.int32)]).reshape(NW, NB, BB)
    ones = jnp.ones((BB,), jnp.float32)
    zeros = jnp.zeros((BB,), jnp.float32)
    z64 = jnp.zeros((BB, D_HID), jnp.float32)
    z128 = jnp.zeros((BB, D_IN), jnp.float32)

    deg2 = _sc_degree(dstp, ones, zeros)
    degT = deg2.T
    g1 = _tc1(degT, x, W1)
    p = _sc_scatter64(g1, srcp, dstp, z64)
    g2 = _tc2(degT, p, g1, b1.reshape(1, -1), W2)
    q = _sc_scatter128(g2, srcp, dstp, z128)
    out = _tc3(degT, q, g2, b2.reshape(1, -1))
    return out


# speedup vs baseline: 20.3007x; 1.4283x over previous
import functools

import jax
import jax.numpy as jnp
from jax import lax
from jax.experimental import pallas as pl
from jax.experimental.pallas import tpu as pltpu
from jax.experimental.pallas import tpu_sc as plsc

N = 10000
E = 320000
D_IN = 128
D_HID = 64

NC = 2
NS = 16
NW = NC * NS
BB = 128
NB = 79
EPT = NB * BB
E_PAD = EPT * NW
N_ACC = 10240
ZSL = N_ACC // BB // NS
R_DUMP = N_ACC // NS
ROW_BLK = 2000


def _mesh():
    return plsc.VectorSubcoreMesh(core_axis_name="c", subcore_axis_name="s")


@functools.partial(
    pl.kernel,
    out_type=jax.ShapeDtypeStruct((NC, N_ACC), jnp.float32),
    mesh=_mesh(),
    scratch_types=[
        pltpu.VMEM((NB, BB), jnp.int32),
        pltpu.VMEM((BB,), jnp.float32),
        pltpu.VMEM((BB,), jnp.float32),
        pltpu.VMEM_SHARED((N_ACC,), jnp.float32),
    ],
)
def _sc_degree(dst_hbm, ones_hbm, zeros_hbm, deg_hbm, dst_v, ones_v, zeros_v, acc):
    cid = lax.axis_index("c")
    sid = lax.axis_index("s")
    wid = sid * NC + cid
    pltpu.sync_copy(dst_hbm.at[wid], dst_v)
    pltpu.sync_copy(ones_hbm, ones_v)
    pltpu.sync_copy(zeros_hbm, zeros_v)
    for k in range(ZSL):
        pltpu.sync_copy(zeros_v, acc.at[pl.ds((sid * ZSL + k) * BB, BB)])
    plsc.subcore_barrier()

    def body(j, carry):
        pltpu.sync_copy(ones_v, acc.at[dst_v.at[j]], add=True)
        return carry

    lax.fori_loop(0, NB, body, 0)
    plsc.subcore_barrier()
    pltpu.sync_copy(acc.at[pl.ds(sid * R_DUMP, R_DUMP)],
                    deg_hbm.at[cid, pl.ds(sid * R_DUMP, R_DUMP)])


def _make_scatter(D):
    @functools.partial(
        pl.kernel,
        out_type=jax.ShapeDtypeStruct((NC, N_ACC, D), jnp.float32),
        mesh=_mesh(),
        compiler_params=pltpu.CompilerParams(use_tc_tiling_on_sc=False),
        scratch_types=[
            pltpu.VMEM((2, BB), jnp.int32),
            pltpu.VMEM((2, BB), jnp.int32),
            pltpu.VMEM((BB, D), jnp.float32),
            pltpu.VMEM((BB, D), jnp.float32),
            pltpu.VMEM_SHARED((N_ACC, D), jnp.float32),
            pltpu.SemaphoreType.DMA,
            pltpu.SemaphoreType.DMA,
            pltpu.SemaphoreType.DMA,
            pltpu.SemaphoreType.DMA,
        ],
    )
    def _scatter(g_hbm, idx_hbm, zrows_hbm, part_hbm,
                 ib0, ib1, buf0, buf1, acc, semg0, semg1, semi0, semi1):
        cid = lax.axis_index("c")
        sid = lax.axis_index("s")
        wid = sid * NC + cid
        pltpu.sync_copy(zrows_hbm, buf0)
        for k in range(ZSL):
            pltpu.sync_copy(buf0, acc.at[pl.ds((sid * ZSL + k) * BB, BB)])
        plsc.subcore_barrier()

        pltpu.sync_copy(idx_hbm.at[wid, 0], ib0)
        pltpu.async_copy(g_hbm.at[ib0.at[0]], buf0, semg0)
        pltpu.async_copy(idx_hbm.at[wid, 1], ib1, semi1)

        def step(j, ib_cur, ib_nxt, buf_cur, buf_nxt,
                 semg_cur, semg_nxt, semi_cur, semi_nxt):
            nxt = j + 1

            @pl.when(nxt < NB)
            def _():
                pltpu.make_async_copy(idx_hbm.at[wid, nxt], ib_nxt, semi_nxt).wait()
                pltpu.async_copy(g_hbm.at[ib_nxt.at[0]], buf_nxt, semg_nxt)

            pltpu.make_async_copy(g_hbm.at[ib_cur.at[0]], buf_cur, semg_cur).wait()
            pltpu.sync_copy(buf_cur, acc.at[ib_cur.at[1]], add=True)

            @pl.when(j + 2 < NB)
            def _():
                pltpu.async_copy(idx_hbm.at[wid, j + 2], ib_cur, semi_cur)

        def body(j, carry):
            @pl.when(j % 2 == 0)
            def _():
                step(j, ib0, ib1, buf0, buf1, semg0, semg1, semi0, semi1)

            @pl.when(j % 2 == 1)
            def _():
                step(j, ib1, ib0, buf1, buf0, semg1, semg0, semi1, semi0)

            return carry

        lax.fori_loop(0, NB, body, 0)
        plsc.subcore_barrier()
        pltpu.sync_copy(acc.at[pl.ds(sid * R_DUMP, R_DUMP)],
                        part_hbm.at[cid, pl.ds(sid * R_DUMP, R_DUMP)])

    return _scatter


_sc_scatter64 = _make_scatter(D_HID)
_sc_scatter128 = _make_scatter(D_IN)


def _dinv_from(degT_ref):
    deg = degT_ref[...].sum(axis=1, keepdims=True) + 1.0
    return lax.rsqrt(jnp.maximum(deg, 1.0))


def _tc1_body(degT_ref, x_ref, W1_ref, g1_ref):
    dinv = _dinv_from(degT_ref)
    g1_ref[...] = jnp.dot(x_ref[...], W1_ref[...],
                          preferred_element_type=jnp.float32) * dinv


def _tc2_body(degT_ref, p_ref, g1_ref, b1_ref, W2_ref, g2_ref):
    dinv = _dinv_from(degT_ref)
    s = p_ref[0] + p_ref[1] + g1_ref[...]
    h = jnp.maximum(dinv * s + b1_ref[...], 0.0)
    g2_ref[...] = jnp.dot(h, W2_ref[...],
                          preferred_element_type=jnp.float32) * dinv


def _tc3_body(degT_ref, q_ref, g2_ref, b2_ref, out_ref):
    dinv = _dinv_from(degT_ref)
    out_ref[...] = dinv * (q_ref[0] + q_ref[1] + g2_ref[...]) + b2_ref[...]


_GRID = (N // ROW_BLK,)

_tc1 = pl.pallas_call(
    _tc1_body,
    grid=_GRID,
    in_specs=[
        pl.BlockSpec((ROW_BLK, 2), lambda i: (i, 0)),
        pl.BlockSpec((ROW_BLK, D_IN), lambda i: (i, 0)),
        pl.BlockSpec((D_IN, D_HID), lambda i: (0, 0)),
    ],
    out_specs=pl.BlockSpec((ROW_BLK, D_HID), lambda i: (i, 0)),
    out_shape=jax.ShapeDtypeStruct((N, D_HID), jnp.float32),
)

_tc2 = pl.pallas_call(
    _tc2_body,
    grid=_GRID,
    in_specs=[
        pl.BlockSpec((ROW_BLK, 2), lambda i: (i, 0)),
        pl.BlockSpec((NC, ROW_BLK, D_HID), lambda i: (0, i, 0)),
        pl.BlockSpec((ROW_BLK, D_HID), lambda i: (i, 0)),
        pl.BlockSpec((1, D_HID), lambda i: (0, 0)),
        pl.BlockSpec((D_HID, D_IN), lambda i: (0, 0)),
    ],
    out_specs=pl.BlockSpec((ROW_BLK, D_IN), lambda i: (i, 0)),
    out_shape=jax.ShapeDtypeStruct((N, D_IN), jnp.float32),
)

_tc3 = pl.pallas_call(
    _tc3_body,
    grid=_GRID,
    in_specs=[
        pl.BlockSpec((ROW_BLK, 2), lambda i: (i, 0)),
        pl.BlockSpec((NC, ROW_BLK, D_IN), lambda i: (0, i, 0)),
        pl.BlockSpec((ROW_BLK, D_IN), lambda i: (i, 0)),
        pl.BlockSpec((1, D_IN), lambda i: (0, 0)),
    ],
    out_specs=pl.BlockSpec((ROW_BLK, D_IN), lambda i: (i, 0)),
    out_shape=jax.ShapeDtypeStruct((N, D_IN), jnp.float32),
)


def kernel(x, edge_index, W1, b1, W2, b2):
    src = edge_index[0].astype(jnp.int32)
    dst = edge_index[1].astype(jnp.int32)
    pad = E_PAD - E
    srcp = jnp.concatenate([src, jnp.zeros((pad,), jnp.int32)]).reshape(NW, NB, BB)
    dstp = jnp.concatenate([dst, jnp.full((pad,), N, jnp.int32)]).reshape(NW, NB, BB)
    ones = jnp.ones((BB,), jnp.float32)
    zeros = jnp.zeros((BB,), jnp.float32)
    z64 = jnp.zeros((BB, D_HID), jnp.float32)
    z128 = jnp.zeros((BB, D_IN), jnp.float32)

    idx = jnp.stack([srcp, dstp], axis=2)

    deg2 = _sc_degree(dstp, ones, zeros)
    degT = deg2.T
    g1 = _tc1(degT, x, W1)
    p = _sc_scatter64(g1, idx, z64)
    g2 = _tc2(degT, p, g1, b1.reshape(1, -1), W2)
    q = _sc_scatter128(g2, idx, z128)
    out = _tc3(degT, q, g2, b2.reshape(1, -1))
    return out
